# baseline (speedup 1.0000x reference)
import jax
import jax.numpy as jnp
from jax.experimental import pallas as pl
from jax.experimental.pallas import tpu as pltpu

B = 512
D = 4096
E = 64
K = 8
IB = 32
NSTEPS = B // IB
NBUF = 4


def _fused_kernel(x_ref, wt_ref, out_hbm, mask_ref, ones_ref,
                  gates_s, maskt_s, ring, sem):
    i = pl.program_id(0)

    @pl.when(i == 0)
    def _router():
        x = x_ref[...]
        wt = wt_ref[...]
        logits = jnp.dot(x, wt, preferred_element_type=jnp.float32)
        m = jnp.max(logits, axis=-1, keepdims=True)
        ex = jnp.exp(logits - m)
        s = jnp.sum(ex, axis=-1, keepdims=True)
        gates = ex / s

        col = jax.lax.broadcasted_iota(jnp.int32, (B, E), 1)
        work = gates
        mask = jnp.zeros((B, E), jnp.float32)
        for _ in range(K):
            mx = jnp.max(work, axis=-1, keepdims=True)
            cand = jnp.where(work == mx, col, E)
            first = jnp.min(cand, axis=-1, keepdims=True)
            pick = col == first
            mask = jnp.where(pick, 1.0, mask)
            work = jnp.where(pick, -1.0, work)

        gates_s[...] = gates
        maskt_s[...] = jnp.transpose(mask)
        mask_ref[...] = mask
        ones_ref[...] = jnp.ones((B, E), jnp.float32)

    slot = jax.lax.rem(i, NBUF)

    def out_copy(step, sl):
        return pltpu.make_async_copy(
            ring.at[sl], out_hbm.at[pl.ds(step * IB, IB)], sem.at[sl])

    @pl.when(i >= NBUF)
    def _reclaim():
        out_copy(i - NBUF, slot).wait()

    mt = maskt_s[...]
    g_blk = gates_s[pl.ds(i * IB, IB), :]
    ring[slot] = g_blk[:, :, None] * mt[None, :, :]
    out_copy(i, slot).start()

    @pl.when(i == NSTEPS - 1)
    def _drain():
        for s in range(NBUF):
            out_copy(NSTEPS - NBUF + s, (i + 1 + s) % NBUF).wait()


def kernel(x, W):
    wt = W.T
    outt, mask, ones = pl.pallas_call(
        _fused_kernel,
        grid=(NSTEPS,),
        in_specs=[
            pl.BlockSpec((B, D), lambda i: (0, 0)),
            pl.BlockSpec((D, E), lambda i: (0, 0)),
        ],
        out_specs=(
            pl.BlockSpec(memory_space=pl.ANY),
            pl.BlockSpec((B, E), lambda i: (0, 0)),
            pl.BlockSpec((B, E), lambda i: (0, 0)),
        ),
        out_shape=(
            jax.ShapeDtypeStruct((B, E, B), jnp.float32),
            jax.ShapeDtypeStruct((B, E), jnp.float32),
            jax.ShapeDtypeStruct((B, E), jnp.float32),
        ),
        scratch_shapes=[
            pltpu.VMEM((B, E), jnp.float32),
            pltpu.VMEM((E, B), jnp.float32),
            pltpu.VMEM((NBUF, IB, E, B), jnp.float32),
            pltpu.SemaphoreType.DMA((NBUF,)),
        ],
    )(x, wt)
    combine = jnp.transpose(outt, (0, 2, 1))
    return (combine, mask, ones)


# speedup vs baseline: 1.0489x; 1.0489x over previous
import jax
import jax.numpy as jnp
from jax.experimental import pallas as pl
from jax.experimental.pallas import tpu as pltpu

B = 512
D = 4096
E = 64
K = 8
IB = 32


def _fused_kernel(x_ref, wt_ref, out_ref, mask_ref, ones_ref,
                  gatest_s, maskt_s):
    i = pl.program_id(0)

    @pl.when(i == 0)
    def _router():
        x = x_ref[...]
        wt = wt_ref[...]
        logits = jnp.dot(x, wt, preferred_element_type=jnp.float32)
        m = jnp.max(logits, axis=-1, keepdims=True)
        ex = jnp.exp(logits - m)
        s = jnp.sum(ex, axis=-1, keepdims=True)
        gates = ex / s

        col = jax.lax.broadcasted_iota(jnp.int32, (B, E), 1)
        work = gates
        mask = jnp.zeros((B, E), jnp.float32)
        for _ in range(K):
            mx = jnp.max(work, axis=-1, keepdims=True)
            cand = jnp.where(work == mx, col, E)
            first = jnp.min(cand, axis=-1, keepdims=True)
            pick = col == first
            mask = jnp.where(pick, 1.0, mask)
            work = jnp.where(pick, -1.0, work)

        gatest_s[...] = gates
        maskt_s[...] = jnp.transpose(mask)
        mask_ref[...] = mask
        ones_ref[...] = jnp.ones((B, E), jnp.float32)

    mt = maskt_s[...]
    g_blk = gatest_s[pl.ds(i * IB, IB), :]
    out_ref[...] = g_blk[:, :, None] * mt[None, :, :]


def kernel(x, W):
    wt = W.T
    outt, mask, ones = pl.pallas_call(
        _fused_kernel,
        grid=(B // IB,),
        in_specs=[
            pl.BlockSpec((B, D), lambda i: (0, 0)),
            pl.BlockSpec((D, E), lambda i: (0, 0)),
        ],
        out_specs=(
            pl.BlockSpec((IB, E, B), lambda i: (i, 0, 0)),
            pl.BlockSpec((B, E), lambda i: (0, 0)),
            pl.BlockSpec((B, E), lambda i: (0, 0)),
        ),
        out_shape=(
            jax.ShapeDtypeStruct((B, E, B), jnp.float32),
            jax.ShapeDtypeStruct((B, E), jnp.float32),
            jax.ShapeDtypeStruct((B, E), jnp.float32),
        ),
        scratch_shapes=[
            pltpu.VMEM((B, E), jnp.float32),
            pltpu.VMEM((E, B), jnp.float32),
        ],
    )(x, wt)
    combine = jnp.transpose(outt, (0, 2, 1))
    return (combine, mask, ones)


# speedup vs baseline: 1.2234x; 1.1663x over previous
import jax
import jax.numpy as jnp
from jax.experimental import pallas as pl
from jax.experimental.pallas import tpu as pltpu

B = 512
D = 4096
E = 64
K = 8
IB = 32


def _fused_kernel(x_ref, wt_ref, out_ref, mask_ref, ones_ref,
                  gatest_s, maskt_s):
    i = pl.program_id(0)

    @pl.when(i == 0)
    def _router():
        x = x_ref[...]
        w = wt_ref[...]
        lt = jax.lax.dot_general(w, x, (((1,), (1,)), ((), ())),
                                 preferred_element_type=jnp.float32)
        m = jnp.max(lt, axis=0, keepdims=True)
        ex = jnp.exp(lt - m)
        s = jnp.sum(ex, axis=0, keepdims=True)
        gt = ex / s

        row = jax.lax.broadcasted_iota(jnp.int32, (E, B), 0)
        work = gt
        mask_t = jnp.zeros((E, B), jnp.float32)
        for _ in range(K):
            mx = jnp.max(work, axis=0, keepdims=True)
            cand = jnp.where(work == mx, row, E)
            first = jnp.min(cand, axis=0, keepdims=True)
            pick = row == first
            mask_t = jnp.where(pick, 1.0, mask_t)
            work = jnp.where(pick, -1.0, work)

        gatest_s[...] = jnp.transpose(gt)
        maskt_s[...] = mask_t
        mask_ref[...] = jnp.transpose(mask_t)
        ones_ref[...] = jnp.ones((B, E), jnp.float32)

    mt = maskt_s[...]
    g_blk = gatest_s[pl.ds(i * IB, IB), :]
    out_ref[...] = g_blk[:, :, None] * mt[None, :, :]


def kernel(x, W):
    outt, mask, ones = pl.pallas_call(
        _fused_kernel,
        grid=(B // IB,),
        in_specs=[
            pl.BlockSpec((B, D), lambda i: (0, 0)),
            pl.BlockSpec((E, D), lambda i: (0, 0)),
        ],
        out_specs=(
            pl.BlockSpec((IB, E, B), lambda i: (i, 0, 0)),
            pl.BlockSpec((B, E), lambda i: (0, 0)),
            pl.BlockSpec((B, E), lambda i: (0, 0)),
        ),
        out_shape=(
            jax.ShapeDtypeStruct((B, E, B), jnp.float32),
            jax.ShapeDtypeStruct((B, E), jnp.float32),
            jax.ShapeDtypeStruct((B, E), jnp.float32),
        ),
        scratch_shapes=[
            pltpu.VMEM((B, E), jnp.float32),
            pltpu.VMEM((E, B), jnp.float32),
        ],
    )(x, W)
    combine = jnp.transpose(outt, (0, 2, 1))
    return (combine, mask, ones)
